# Initial kernel scaffold; baseline (speedup 1.0000x reference)
import functools

import jax
import jax.numpy as jnp
from jax import lax
from jax.experimental import pallas as pl
from jax.experimental.pallas import tpu as pltpu
from jax.experimental.pallas import tpu_sc as plsc

N = 10000
E = 160000
D_IN = 128
D_EDGE = 4
D_H = 32

NC = 2
NS = 16
NW = NC * NS
PER_W = E // NW
CHUNK = 128
NFULL = PER_W // CHUNK
TAIL = PER_W - NFULL * CHUNK
ROWS_PER_TILE = N // NS

_MESH = plsc.VectorSubcoreMesh(core_axis_name="c", subcore_axis_name="s")


@functools.partial(
    pl.kernel,
    out_type=jax.ShapeDtypeStruct((E, D_IN), jnp.float32),
    mesh=_MESH,
    scratch_types=[
        pltpu.VMEM((CHUNK,), jnp.int32),
        pltpu.VMEM((CHUNK, D_IN), jnp.float32),
        pltpu.VMEM((TAIL,), jnp.int32),
        pltpu.VMEM((TAIL, D_IN), jnp.float32),
        pltpu.SemaphoreType.DMA,
    ],
)
def _gather_rows(table_hbm, idx_hbm, out_hbm, idx_v, rows_v, idx_t, rows_t, sem):
    wid = lax.axis_index("s") * NC + lax.axis_index("c")
    base = wid * PER_W

    def body(i, carry):
        off = base + i * CHUNK
        pltpu.sync_copy(idx_hbm.at[pl.ds(off, CHUNK)], idx_v)
        pltpu.async_copy(table_hbm.at[idx_v], rows_v, sem).wait()
        pltpu.sync_copy(rows_v, out_hbm.at[pl.ds(off, CHUNK)])
        return carry

    lax.fori_loop(0, NFULL, body, 0)

    off = base + NFULL * CHUNK
    pltpu.sync_copy(idx_hbm.at[pl.ds(off, TAIL)], idx_t)
    pltpu.async_copy(table_hbm.at[idx_t], rows_t, sem).wait()
    pltpu.sync_copy(rows_t, out_hbm.at[pl.ds(off, TAIL)])


@functools.partial(
    pl.kernel,
    out_type=jax.ShapeDtypeStruct((NC, N, D_H), jnp.float32),
    mesh=_MESH,
    scratch_types=[
        pltpu.VMEM((CHUNK,), jnp.int32),
        pltpu.VMEM((CHUNK, D_H), jnp.float32),
        pltpu.VMEM((TAIL,), jnp.int32),
        pltpu.VMEM((TAIL, D_H), jnp.float32),
        pltpu.VMEM_SHARED((N, D_H), jnp.float32),
    ],
)
def _scatter_add(m_hbm, idx_hbm, zeros_hbm, out_hbm, idx_v, m_v, idx_t, m_t, acc_sh):
    cid = lax.axis_index("c")
    sid = lax.axis_index("s")
    wid = sid * NC + cid
    base = wid * PER_W
    stripe = pl.ds(sid * ROWS_PER_TILE, ROWS_PER_TILE)

    pltpu.sync_copy(zeros_hbm.at[stripe], acc_sh.at[stripe])
    plsc.subcore_barrier()

    def body(i, carry):
        off = base + i * CHUNK
        pltpu.sync_copy(idx_hbm.at[pl.ds(off, CHUNK)], idx_v)
        pltpu.sync_copy(m_hbm.at[pl.ds(off, CHUNK)], m_v)
        pltpu.sync_copy(m_v, acc_sh.at[idx_v], add=True)
        return carry

    lax.fori_loop(0, NFULL, body, 0)

    off = base + NFULL * CHUNK
    pltpu.sync_copy(idx_hbm.at[pl.ds(off, TAIL)], idx_t)
    pltpu.sync_copy(m_hbm.at[pl.ds(off, TAIL)], m_t)
    pltpu.sync_copy(m_t, acc_sh.at[idx_t], add=True)

    plsc.subcore_barrier()
    pltpu.sync_copy(acc_sh.at[stripe], out_hbm.at[cid, stripe])


BE = 4000


def _edge_body(xg_ref, ea_ref, w1_ref, wl_ref, out_ref):
    t = jnp.dot(xg_ref[...], w1_ref[...], preferred_element_type=jnp.float32)
    ea = ea_ref[...]
    m = ea[:, 0:1] * t[:, 0:32]
    m += ea[:, 1:2] * t[:, 32:64]
    m += ea[:, 2:3] * t[:, 64:96]
    m += ea[:, 3:4] * t[:, 96:128]
    m = m * jax.nn.sigmoid(m)
    out_ref[...] = jnp.dot(m, wl_ref[...], preferred_element_type=jnp.float32)


def _edge_stage(xg, ea, w1, wl):
    return pl.pallas_call(
        _edge_body,
        grid=(E // BE,),
        in_specs=[
            pl.BlockSpec((BE, D_IN), lambda i: (i, 0)),
            pl.BlockSpec((BE, D_EDGE), lambda i: (i, 0)),
            pl.BlockSpec((D_IN, D_EDGE * D_H), lambda i: (0, 0)),
            pl.BlockSpec((D_H, D_H), lambda i: (0, 0)),
        ],
        out_specs=pl.BlockSpec((BE, D_H), lambda i: (i, 0)),
        out_shape=jax.ShapeDtypeStruct((E, D_H), jnp.float32),
    )(xg, ea, w1, wl)


BN = 1000


def _update_body(x_ref, p0_ref, p1_ref, w2_ref, wl2_ref, out_ref):
    x = x_ref[...]
    agg = p0_ref[...] + p1_ref[...]
    t = jnp.dot(x, w2_ref[...], preferred_element_type=jnp.float32)
    u = agg[:, 0:1] * t[:, 0:32]
    for h in range(1, D_H):
        u += agg[:, h:h + 1] * t[:, h * 32:(h + 1) * 32]
    u = u * jax.nn.sigmoid(u)
    out_ref[...] = x + jnp.dot(u, wl2_ref[...], preferred_element_type=jnp.float32)


def _update_stage(x, p0, p1, w2, wl2):
    return pl.pallas_call(
        _update_body,
        grid=(N // BN,),
        in_specs=[
            pl.BlockSpec((BN, D_IN), lambda i: (i, 0)),
            pl.BlockSpec((BN, D_H), lambda i: (i, 0)),
            pl.BlockSpec((BN, D_H), lambda i: (i, 0)),
            pl.BlockSpec((D_IN, D_H * D_H), lambda i: (0, 0)),
            pl.BlockSpec((D_H, D_IN), lambda i: (0, 0)),
        ],
        out_specs=pl.BlockSpec((BN, D_IN), lambda i: (i, 0)),
        out_shape=jax.ShapeDtypeStruct((N, D_IN), jnp.float32),
    )(x, p0, p1, w2, wl2)


def kernel(node_features, edge_index, edge_attr_e3nn, node_attr_scalar_raw,
           W_tp_msg, W_lin_msg, W_tp_upd, W_lin_upd):
    del node_attr_scalar_raw
    row = edge_index[0].astype(jnp.int32)
    col = edge_index[1].astype(jnp.int32)
    w1 = W_tp_msg.reshape(D_IN, D_EDGE * D_H)
    w2 = W_tp_upd.reshape(D_IN, D_H * D_H)
    zeros = jnp.zeros((N, D_H), jnp.float32)

    xg = _gather_rows(node_features, col)
    m = _edge_stage(xg, edge_attr_e3nn, w1, W_lin_msg)
    partials = _scatter_add(m, row, zeros)
    return _update_stage(node_features, partials[0], partials[1], w2, W_lin_upd)


# speedup vs baseline: 1.9581x; 1.9581x over previous
import functools

import jax
import jax.numpy as jnp
from jax import lax
from jax.experimental import pallas as pl
from jax.experimental.pallas import tpu as pltpu
from jax.experimental.pallas import tpu_sc as plsc

N = 10000
E = 160000
D_IN = 128
D_EDGE = 4
D_H = 32

NC = 2
NS = 16
NW = NC * NS
PER_W = E // NW
CHUNK = 128
NFULL = PER_W // CHUNK
TAIL = PER_W - NFULL * CHUNK
NPAD = 10240
ROWS_PER_TILE = NPAD // NS

@functools.lru_cache(maxsize=None)
def _sc_kernels():
  _MESH = plsc.VectorSubcoreMesh(core_axis_name="c", subcore_axis_name="s",
                                 num_cores=NC, num_subcores=NS)

  @functools.partial(
    pl.kernel,
    out_type=jax.ShapeDtypeStruct((E, D_IN), jnp.float32),
    mesh=_MESH,
    scratch_types=[
        pltpu.VMEM((CHUNK,), jnp.int32),
        pltpu.VMEM((CHUNK, D_IN), jnp.float32),
        pltpu.VMEM((TAIL,), jnp.int32),
        pltpu.VMEM((TAIL, D_IN), jnp.float32),
        pltpu.SemaphoreType.DMA,
    ],
)
  def _gather_rows(table_hbm, idx_hbm, out_hbm, idx_v, rows_v, idx_t, rows_t, sem):
    wid = lax.axis_index("s") * NC + lax.axis_index("c")
    base = wid * PER_W

    def body(i, carry):
        off = base + i * CHUNK
        pltpu.sync_copy(idx_hbm.at[pl.ds(off, CHUNK)], idx_v)
        pltpu.async_copy(table_hbm.at[idx_v], rows_v, sem).wait()
        pltpu.sync_copy(rows_v, out_hbm.at[pl.ds(off, CHUNK)])
        return carry

    lax.fori_loop(0, NFULL, body, 0)

    off = base + NFULL * CHUNK
    pltpu.sync_copy(idx_hbm.at[pl.ds(off, TAIL)], idx_t)
    pltpu.async_copy(table_hbm.at[idx_t], rows_t, sem).wait()
    pltpu.sync_copy(rows_t, out_hbm.at[pl.ds(off, TAIL)])


  @functools.partial(
    pl.kernel,
    out_type=jax.ShapeDtypeStruct((NC, NPAD, D_IN), jnp.float32),
    mesh=_MESH,
    scratch_types=[
        pltpu.VMEM((CHUNK,), jnp.int32),
        pltpu.VMEM((CHUNK, D_IN), jnp.float32),
        pltpu.VMEM((TAIL,), jnp.int32),
        pltpu.VMEM((TAIL, D_IN), jnp.float32),
        pltpu.VMEM_SHARED((NPAD, D_IN), jnp.float32),
    ],
)
  def _scatter_add(m_hbm, idx_hbm, zeros_hbm, out_hbm, idx_v, m_v, idx_t, m_t, acc_sh):
    cid = lax.axis_index("c")
    sid = lax.axis_index("s")
    wid = sid * NC + cid
    base = wid * PER_W
    stripe = pl.ds(sid * ROWS_PER_TILE, ROWS_PER_TILE)

    pltpu.sync_copy(zeros_hbm.at[stripe], acc_sh.at[stripe])
    plsc.subcore_barrier()

    def body(i, carry):
        off = base + i * CHUNK
        pltpu.sync_copy(idx_hbm.at[pl.ds(off, CHUNK)], idx_v)
        pltpu.sync_copy(m_hbm.at[pl.ds(off, CHUNK)], m_v)
        pltpu.sync_copy(m_v, acc_sh.at[idx_v], add=True)
        return carry

    lax.fori_loop(0, NFULL, body, 0)

    off = base + NFULL * CHUNK
    pltpu.sync_copy(idx_hbm.at[pl.ds(off, TAIL)], idx_t)
    pltpu.sync_copy(m_hbm.at[pl.ds(off, TAIL)], m_t)
    pltpu.sync_copy(m_t, acc_sh.at[idx_t], add=True)

    plsc.subcore_barrier()
    pltpu.sync_copy(acc_sh.at[stripe], out_hbm.at[cid, stripe])

  return _gather_rows, _scatter_add


BE = 4000


def _edge_body(xg_ref, ea_ref, w1_ref, wl_ref, out_ref):
    t = jnp.dot(xg_ref[...], w1_ref[...], preferred_element_type=jnp.float32)
    ea = ea_ref[...]
    m = ea[:, 0:1] * t[:, 0:32]
    m += ea[:, 1:2] * t[:, 32:64]
    m += ea[:, 2:3] * t[:, 64:96]
    m += ea[:, 3:4] * t[:, 96:128]
    m = m * jax.nn.sigmoid(m)
    m = jnp.dot(m, wl_ref[...], preferred_element_type=jnp.float32)
    out_ref[...] = jnp.concatenate(
        [m, jnp.zeros((m.shape[0], D_IN - D_H), jnp.float32)], axis=1)


def _edge_stage(xg, ea, w1, wl):
    return pl.pallas_call(
        _edge_body,
        grid=(E // BE,),
        in_specs=[
            pl.BlockSpec((BE, D_IN), lambda i: (i, 0)),
            pl.BlockSpec((BE, D_EDGE), lambda i: (i, 0)),
            pl.BlockSpec((D_IN, D_EDGE * D_H), lambda i: (0, 0)),
            pl.BlockSpec((D_H, D_H), lambda i: (0, 0)),
        ],
        out_specs=pl.BlockSpec((BE, D_IN), lambda i: (i, 0)),
        out_shape=jax.ShapeDtypeStruct((E, D_IN), jnp.float32),
    )(xg, ea, w1, wl)


BN = 1000


def _update_body(x_ref, p0_ref, p1_ref, w2_ref, wl2_ref, out_ref):
    x = x_ref[...]
    agg = p0_ref[:, :D_H] + p1_ref[:, :D_H]
    t = jnp.dot(x, w2_ref[...], preferred_element_type=jnp.float32)
    u = agg[:, 0:1] * t[:, 0:32]
    for h in range(1, D_H):
        u += agg[:, h:h + 1] * t[:, h * 32:(h + 1) * 32]
    u = u * jax.nn.sigmoid(u)
    out_ref[...] = x + jnp.dot(u, wl2_ref[...], preferred_element_type=jnp.float32)


def _update_stage(x, p0, p1, w2, wl2):
    return pl.pallas_call(
        _update_body,
        grid=(N // BN,),
        in_specs=[
            pl.BlockSpec((BN, D_IN), lambda i: (i, 0)),
            pl.BlockSpec((BN, D_IN), lambda i: (i, 0)),
            pl.BlockSpec((BN, D_IN), lambda i: (i, 0)),
            pl.BlockSpec((D_IN, D_H * D_H), lambda i: (0, 0)),
            pl.BlockSpec((D_H, D_IN), lambda i: (0, 0)),
        ],
        out_specs=pl.BlockSpec((BN, D_IN), lambda i: (i, 0)),
        out_shape=jax.ShapeDtypeStruct((N, D_IN), jnp.float32),
    )(x, p0, p1, w2, wl2)


def kernel(node_features, edge_index, edge_attr_e3nn, node_attr_scalar_raw,
           W_tp_msg, W_lin_msg, W_tp_upd, W_lin_upd):
    del node_attr_scalar_raw
    row = edge_index[0].astype(jnp.int32)
    col = edge_index[1].astype(jnp.int32)
    w1 = W_tp_msg.reshape(D_IN, D_EDGE * D_H)
    w2 = W_tp_upd.reshape(D_IN, D_H * D_H)
    zeros = jnp.zeros((NPAD, D_IN), jnp.float32)

    _gather_rows, _scatter_add = _sc_kernels()
    xg = _gather_rows(node_features, col)
    m = _edge_stage(xg, edge_attr_e3nn, w1, W_lin_msg)
    partials = _scatter_add(m, row, zeros)
    return _update_stage(node_features, partials[0, :N], partials[1, :N], w2, W_lin_upd)


# speedup vs baseline: 1.9846x; 1.0135x over previous
import functools

import jax
import jax.numpy as jnp
from jax import lax
from jax.experimental import pallas as pl
from jax.experimental.pallas import tpu as pltpu
from jax.experimental.pallas import tpu_sc as plsc

N = 10000
E = 160000
D_IN = 128
D_EDGE = 4
D_H = 32

NC = 2
NS = 16
NW = NC * NS
PER_W = E // NW
CHUNK = 128
NFULL = PER_W // CHUNK
TAIL = PER_W - NFULL * CHUNK
NPAD = 10240
ROWS_PER_TILE = NPAD // NS
L = 16


@functools.lru_cache(maxsize=None)
def _sc_kernels():
  mesh = plsc.VectorSubcoreMesh(core_axis_name="c", subcore_axis_name="s",
                                num_cores=NC, num_subcores=NS)

  @functools.partial(
    pl.kernel,
    out_type=jax.ShapeDtypeStruct((NC, NPAD, D_IN), jnp.float32),
    mesh=mesh,
    scratch_types=[
        pltpu.VMEM((CHUNK,), jnp.int32),
        pltpu.VMEM((CHUNK,), jnp.int32),
        pltpu.VMEM((CHUNK * D_EDGE + L,), jnp.float32),
        pltpu.VMEM((CHUNK, D_IN), jnp.float32),
        pltpu.VMEM((CHUNK, D_IN), jnp.float32),
        pltpu.VMEM_SHARED((NPAD, D_IN), jnp.float32),
        pltpu.SemaphoreType.DMA,
    ],
  )
  def _edge_fused(y_hbm, col_hbm, row_hbm, ea_hbm, zeros_hbm, out_hbm,
                  cidx_v, ridx_v, ea_v, yg_v, m_v, acc_sh, sem):
    cid = lax.axis_index("c")
    sid = lax.axis_index("s")
    wid = sid * NC + cid
    base = wid * PER_W
    stripe = pl.ds(sid * ROWS_PER_TILE, ROWS_PER_TILE)

    pltpu.sync_copy(zeros_hbm.at[stripe], acc_sh.at[stripe])

    zed = jnp.zeros((L,), jnp.float32)

    def zrow(r, carry):
        for sj in range(D_IN // L):
            m_v[r, pl.ds(sj * L, L)] = zed
        return carry

    lax.fori_loop(0, CHUNK, zrow, 0)
    plsc.subcore_barrier()

    def compute_edge(e, carry):
        av = ea_v[pl.ds(D_EDGE * e, L)]
        a0 = av[0]
        a1 = av[1]
        a2 = av[2]
        a3 = av[3]
        for q in range(D_H // L):
            v = (a0 * yg_v[e, pl.ds(q * L, L)]
                 + a1 * yg_v[e, pl.ds(32 + q * L, L)]
                 + a2 * yg_v[e, pl.ds(64 + q * L, L)]
                 + a3 * yg_v[e, pl.ds(96 + q * L, L)])
            s = 1.0 / (1.0 + jnp.exp(-v))
            m_v[e, pl.ds(q * L, L)] = v * s
        return carry

    def chunk_body(i, carry):
        off = base + i * CHUNK
        pltpu.sync_copy(col_hbm.at[pl.ds(off, CHUNK)], cidx_v)
        pltpu.sync_copy(row_hbm.at[pl.ds(off, CHUNK)], ridx_v)
        pltpu.sync_copy(ea_hbm.at[pl.ds(off * D_EDGE, CHUNK * D_EDGE)],
                        ea_v.at[pl.ds(0, CHUNK * D_EDGE)])
        pltpu.async_copy(y_hbm.at[cidx_v], yg_v, sem).wait()
        lax.fori_loop(0, CHUNK, compute_edge, 0)
        pltpu.sync_copy(m_v, acc_sh.at[ridx_v], add=True)
        return carry

    lax.fori_loop(0, NFULL, chunk_body, 0)

    off = base + NFULL * CHUNK
    pltpu.sync_copy(col_hbm.at[pl.ds(off, TAIL)], cidx_v.at[pl.ds(0, TAIL)])
    pltpu.sync_copy(row_hbm.at[pl.ds(off, TAIL)], ridx_v.at[pl.ds(0, TAIL)])
    pltpu.sync_copy(ea_hbm.at[pl.ds(off * D_EDGE, TAIL * D_EDGE)],
                    ea_v.at[pl.ds(0, TAIL * D_EDGE)])
    pltpu.async_copy(y_hbm.at[cidx_v.at[pl.ds(0, TAIL)]],
                     yg_v.at[pl.ds(0, TAIL)], sem).wait()
    lax.fori_loop(0, TAIL, compute_edge, 0)
    pltpu.sync_copy(m_v.at[pl.ds(0, TAIL)],
                    acc_sh.at[ridx_v.at[pl.ds(0, TAIL)]], add=True)

    plsc.subcore_barrier()
    pltpu.sync_copy(acc_sh.at[stripe], out_hbm.at[cid, stripe])

  return _edge_fused


BY = 2000


def _prep_body(x_ref, w1_ref, y_ref):
    y_ref[...] = jnp.dot(x_ref[...], w1_ref[...],
                         preferred_element_type=jnp.float32)


def _prep_stage(x, w1):
    return pl.pallas_call(
        _prep_body,
        grid=(N // BY,),
        in_specs=[
            pl.BlockSpec((BY, D_IN), lambda i: (i, 0)),
            pl.BlockSpec((D_IN, D_EDGE * D_H), lambda i: (0, 0)),
        ],
        out_specs=pl.BlockSpec((BY, D_EDGE * D_H), lambda i: (i, 0)),
        out_shape=jax.ShapeDtypeStruct((N, D_EDGE * D_H), jnp.float32),
    )(x, w1)


BN = 1000


def _update_body(x_ref, p0_ref, p1_ref, wl_ref, w2_ref, wl2_ref, out_ref):
    x = x_ref[...]
    agg = p0_ref[:, :D_H] + p1_ref[:, :D_H]
    agg = jnp.dot(agg, wl_ref[...], preferred_element_type=jnp.float32)
    t = jnp.dot(x, w2_ref[...], preferred_element_type=jnp.float32)
    u = agg[:, 0:1] * t[:, 0:32]
    for h in range(1, D_H):
        u += agg[:, h:h + 1] * t[:, h * 32:(h + 1) * 32]
    u = u * jax.nn.sigmoid(u)
    out_ref[...] = x + jnp.dot(u, wl2_ref[...], preferred_element_type=jnp.float32)


def _update_stage(x, p0, p1, wl, w2, wl2):
    return pl.pallas_call(
        _update_body,
        grid=(N // BN,),
        in_specs=[
            pl.BlockSpec((BN, D_IN), lambda i: (i, 0)),
            pl.BlockSpec((BN, D_IN), lambda i: (i, 0)),
            pl.BlockSpec((BN, D_IN), lambda i: (i, 0)),
            pl.BlockSpec((D_H, D_H), lambda i: (0, 0)),
            pl.BlockSpec((D_IN, D_H * D_H), lambda i: (0, 0)),
            pl.BlockSpec((D_H, D_IN), lambda i: (0, 0)),
        ],
        out_specs=pl.BlockSpec((BN, D_IN), lambda i: (i, 0)),
        out_shape=jax.ShapeDtypeStruct((N, D_IN), jnp.float32),
    )(x, p0, p1, wl, w2, wl2)


def kernel(node_features, edge_index, edge_attr_e3nn, node_attr_scalar_raw,
           W_tp_msg, W_lin_msg, W_tp_upd, W_lin_upd):
    del node_attr_scalar_raw
    row = edge_index[0].astype(jnp.int32)
    col = edge_index[1].astype(jnp.int32)
    w1 = W_tp_msg.reshape(D_IN, D_EDGE * D_H)
    w2 = W_tp_upd.reshape(D_IN, D_H * D_H)
    ea1d = edge_attr_e3nn.reshape(-1)
    zeros = jnp.zeros((NPAD, D_IN), jnp.float32)

    y = _prep_stage(node_features, w1)
    _edge_fused = _sc_kernels()
    partials = _edge_fused(y, col, row, ea1d, zeros)
    return _update_stage(node_features, partials[0, :N], partials[1, :N],
                         W_lin_msg, w2, W_lin_upd)


# speedup vs baseline: 2.4362x; 1.2276x over previous
import functools

import jax
import jax.numpy as jnp
from jax import lax
from jax.experimental import pallas as pl
from jax.experimental.pallas import tpu as pltpu
from jax.experimental.pallas import tpu_sc as plsc

N = 10000
E = 160000
D_IN = 128
D_EDGE = 4
D_H = 32

NC = 2
NS = 16
NW = NC * NS
PER_W = E // NW
CHUNK = 128
NFULL = PER_W // CHUNK
TAIL = PER_W - NFULL * CHUNK
NPAD = 10240
ROWS_PER_TILE = NPAD // NS
L = 16


@functools.lru_cache(maxsize=None)
def _sc_kernels():
  mesh = plsc.VectorSubcoreMesh(core_axis_name="c", subcore_axis_name="s",
                                num_cores=NC, num_subcores=NS)

  @functools.partial(
    pl.kernel,
    out_type=jax.ShapeDtypeStruct((NC, NPAD, D_IN), jnp.float32),
    mesh=mesh,
    scratch_types=[
        pltpu.VMEM((CHUNK,), jnp.int32),
        pltpu.VMEM((CHUNK,), jnp.int32),
        pltpu.VMEM((CHUNK * D_EDGE + L,), jnp.float32),
        pltpu.VMEM((CHUNK, D_IN), jnp.float32),
        pltpu.VMEM((CHUNK, D_IN), jnp.float32),
        pltpu.VMEM_SHARED((NPAD, D_IN), jnp.float32),
        pltpu.SemaphoreType.DMA,
    ],
  )
  def _edge_fused(y_hbm, col_hbm, row_hbm, ea_hbm, zeros_hbm, out_hbm,
                  cidx_v, ridx_v, ea_v, yg_v, m_v, acc_sh, sem):
    cid = lax.axis_index("c")
    sid = lax.axis_index("s")
    wid = sid * NC + cid
    base = wid * PER_W
    stripe = pl.ds(sid * ROWS_PER_TILE, ROWS_PER_TILE)

    pltpu.sync_copy(zeros_hbm.at[stripe], acc_sh.at[stripe])

    zed = jnp.zeros((L,), jnp.float32)

    def zrow(r, carry):
        for sj in range(D_IN // L):
            m_v[r, pl.ds(sj * L, L)] = zed
        return carry

    lax.fori_loop(0, CHUNK, zrow, 0)
    plsc.subcore_barrier()

    def _edge_math(e):
        av = ea_v[pl.ds(D_EDGE * e, L)]
        a0 = av[0]
        a1 = av[1]
        a2 = av[2]
        a3 = av[3]
        for q in range(D_H // L):
            v = (a0 * yg_v[e, pl.ds(q * L, L)]
                 + a1 * yg_v[e, pl.ds(32 + q * L, L)]
                 + a2 * yg_v[e, pl.ds(64 + q * L, L)]
                 + a3 * yg_v[e, pl.ds(96 + q * L, L)])
            s = 1.0 / (1.0 + jnp.exp(-v))
            m_v[e, pl.ds(q * L, L)] = v * s

    def compute_edges(n):
        plsc.parallel_loop(0, n, 1, unroll=4)(_edge_math)

    def chunk_body(i, carry):
        off = base + i * CHUNK
        pltpu.sync_copy(col_hbm.at[pl.ds(off, CHUNK)], cidx_v)
        pltpu.sync_copy(row_hbm.at[pl.ds(off, CHUNK)], ridx_v)
        pltpu.sync_copy(ea_hbm.at[pl.ds(off * D_EDGE, CHUNK * D_EDGE)],
                        ea_v.at[pl.ds(0, CHUNK * D_EDGE)])
        pltpu.async_copy(y_hbm.at[cidx_v], yg_v, sem).wait()
        compute_edges(CHUNK)
        pltpu.sync_copy(m_v, acc_sh.at[ridx_v], add=True)
        return carry

    lax.fori_loop(0, NFULL, chunk_body, 0)

    off = base + NFULL * CHUNK
    pltpu.sync_copy(col_hbm.at[pl.ds(off, TAIL)], cidx_v.at[pl.ds(0, TAIL)])
    pltpu.sync_copy(row_hbm.at[pl.ds(off, TAIL)], ridx_v.at[pl.ds(0, TAIL)])
    pltpu.sync_copy(ea_hbm.at[pl.ds(off * D_EDGE, TAIL * D_EDGE)],
                    ea_v.at[pl.ds(0, TAIL * D_EDGE)])
    pltpu.async_copy(y_hbm.at[cidx_v.at[pl.ds(0, TAIL)]],
                     yg_v.at[pl.ds(0, TAIL)], sem).wait()
    compute_edges(TAIL)
    pltpu.sync_copy(m_v.at[pl.ds(0, TAIL)],
                    acc_sh.at[ridx_v.at[pl.ds(0, TAIL)]], add=True)

    plsc.subcore_barrier()
    pltpu.sync_copy(acc_sh.at[stripe], out_hbm.at[cid, stripe])

  return _edge_fused


BY = 2000


def _prep_body(x_ref, w1_ref, y_ref):
    y_ref[...] = jnp.dot(x_ref[...], w1_ref[...],
                         preferred_element_type=jnp.float32)


def _prep_stage(x, w1):
    return pl.pallas_call(
        _prep_body,
        grid=(N // BY,),
        in_specs=[
            pl.BlockSpec((BY, D_IN), lambda i: (i, 0)),
            pl.BlockSpec((D_IN, D_EDGE * D_H), lambda i: (0, 0)),
        ],
        out_specs=pl.BlockSpec((BY, D_EDGE * D_H), lambda i: (i, 0)),
        out_shape=jax.ShapeDtypeStruct((N, D_EDGE * D_H), jnp.float32),
    )(x, w1)


BN = 1000


def _update_body(x_ref, p0_ref, p1_ref, wl_ref, w2_ref, wl2_ref, out_ref):
    x = x_ref[...]
    agg = p0_ref[:, :D_H] + p1_ref[:, :D_H]
    agg = jnp.dot(agg, wl_ref[...], preferred_element_type=jnp.float32)
    t = jnp.dot(x, w2_ref[...], preferred_element_type=jnp.float32)
    u = agg[:, 0:1] * t[:, 0:32]
    for h in range(1, D_H):
        u += agg[:, h:h + 1] * t[:, h * 32:(h + 1) * 32]
    u = u * jax.nn.sigmoid(u)
    out_ref[...] = x + jnp.dot(u, wl2_ref[...], preferred_element_type=jnp.float32)


def _update_stage(x, p0, p1, wl, w2, wl2):
    return pl.pallas_call(
        _update_body,
        grid=(N // BN,),
        in_specs=[
            pl.BlockSpec((BN, D_IN), lambda i: (i, 0)),
            pl.BlockSpec((BN, D_IN), lambda i: (i, 0)),
            pl.BlockSpec((BN, D_IN), lambda i: (i, 0)),
            pl.BlockSpec((D_H, D_H), lambda i: (0, 0)),
            pl.BlockSpec((D_IN, D_H * D_H), lambda i: (0, 0)),
            pl.BlockSpec((D_H, D_IN), lambda i: (0, 0)),
        ],
        out_specs=pl.BlockSpec((BN, D_IN), lambda i: (i, 0)),
        out_shape=jax.ShapeDtypeStruct((N, D_IN), jnp.float32),
    )(x, p0, p1, wl, w2, wl2)


def kernel(node_features, edge_index, edge_attr_e3nn, node_attr_scalar_raw,
           W_tp_msg, W_lin_msg, W_tp_upd, W_lin_upd):
    del node_attr_scalar_raw
    row = edge_index[0].astype(jnp.int32)
    col = edge_index[1].astype(jnp.int32)
    w1 = W_tp_msg.reshape(D_IN, D_EDGE * D_H)
    w2 = W_tp_upd.reshape(D_IN, D_H * D_H)
    ea1d = edge_attr_e3nn.reshape(-1)
    zeros = jnp.zeros((NPAD, D_IN), jnp.float32)

    y = _prep_stage(node_features, w1)
    _edge_fused = _sc_kernels()
    partials = _edge_fused(y, col, row, ea1d, zeros)
    return _update_stage(node_features, partials[0, :N], partials[1, :N],
                         W_lin_msg, w2, W_lin_upd)
